# baseline (speedup 1.0000x reference)
import functools

import jax
import jax.numpy as jnp
from jax import lax
from jax.experimental import pallas as pl
from jax.experimental.pallas import tpu as pltpu
from jax.experimental.pallas import tpu_sc as plsc

NC = 2
NS = 16
NW = NC * NS
CHUNK = 128
LANES = 16

RATE_SCALE = 4.0
RATE_LO = 1.0


@functools.lru_cache(maxsize=None)
def _build(B, D, V):
    b_per_w = B // NW
    n_chunks = b_per_w // CHUNK
    n_groups = b_per_w // LANES

    mesh = plsc.VectorSubcoreMesh(core_axis_name="c", subcore_axis_name="s")

    @functools.partial(
        pl.kernel,
        mesh=mesh,
        out_type=jax.ShapeDtypeStruct((B,), jnp.float32),
        compiler_params=pltpu.CompilerParams(
            needs_layout_passes=False, use_tc_tiling_on_sc=False),
        scratch_types=[
            pltpu.VMEM((b_per_w,), jnp.int32),
            pltpu.VMEM((b_per_w,), jnp.int32),
            pltpu.VMEM((D * b_per_w,), jnp.float32),
            pltpu.VMEM((D * b_per_w,), jnp.float32),
            pltpu.VMEM((b_per_w,), jnp.float32),
            pltpu.VMEM((b_per_w,), jnp.float32),
            pltpu.VMEM((b_per_w,), jnp.float32),
            pltpu.SemaphoreType.DMA,
            pltpu.SemaphoreType.DMA,
            pltpu.SemaphoreType.DMA,
        ],
    )
    def mf_kernel(uidx_hbm, iidx_hbm, uT_hbm, iT_hbm, ub_hbm, ib_hbm,
                  out_hbm, uidx_v, iidx_v, u_dst, v_dst, ub_v, ib_v,
                  out_v, sem_u, sem_v, sem_b):
        wid = lax.axis_index("c") * NS + lax.axis_index("s")
        base_e = wid * b_per_w

        pltpu.sync_copy(uidx_hbm.at[pl.ds(base_e, b_per_w)], uidx_v)
        pltpu.sync_copy(iidx_hbm.at[pl.ds(base_e, b_per_w)], iidx_v)

        for j in range(n_chunks):
            s = pl.ds(j * CHUNK, CHUNK)
            pltpu.async_copy(ub_hbm.at[uidx_v.at[s]], ub_v.at[s], sem_b)
            pltpu.async_copy(ib_hbm.at[iidx_v.at[s]], ib_v.at[s], sem_b)

        for c in range(D):
            for j in range(n_chunks):
                s = pl.ds(j * CHUNK, CHUNK)
                d = pl.ds(c * b_per_w + j * CHUNK, CHUNK)
                pltpu.async_copy(
                    uT_hbm.at[c].at[uidx_v.at[s]], u_dst.at[d], sem_u)
                pltpu.async_copy(
                    iT_hbm.at[c].at[iidx_v.at[s]], v_dst.at[d], sem_v)

        pltpu.make_async_copy(
            uT_hbm.at[0].at[pl.ds(0, D * b_per_w)], u_dst, sem_u).wait()
        pltpu.make_async_copy(
            iT_hbm.at[0].at[pl.ds(0, D * b_per_w)], v_dst, sem_v).wait()
        pltpu.make_async_copy(ub_hbm.at[pl.ds(0, b_per_w)], ub_v, sem_b).wait()
        pltpu.make_async_copy(ib_hbm.at[pl.ds(0, b_per_w)], ib_v, sem_b).wait()

        def group_body(g, carry):
            base = g * LANES
            acc = ub_v[pl.ds(base, LANES)] + ib_v[pl.ds(base, LANES)]
            for c in range(D):
                acc = acc + (u_dst[pl.ds(c * b_per_w + base, LANES)]
                             * v_dst[pl.ds(c * b_per_w + base, LANES)])
            pred = RATE_SCALE / (1.0 + jnp.exp(-acc)) + RATE_LO
            out_v[pl.ds(base, LANES)] = pred
            return carry

        lax.fori_loop(0, n_groups, group_body, 0)

        pltpu.sync_copy(out_v, out_hbm.at[pl.ds(base_e, b_per_w)])

    return mf_kernel


def kernel(X, user_embeddings, user_bias, item_embeddings, item_bias):
    B = X.shape[0]
    V, D = user_embeddings.shape
    uidx = X[:, 0].astype(jnp.int32)
    iidx = X[:, 1].astype(jnp.int32)
    ub = user_bias.reshape(-1)
    ib = item_bias.reshape(-1)
    out = _build(B, D, V)(uidx, iidx, user_embeddings.T, item_embeddings.T,
                          ub, ib)
    return out.reshape(B, 1)


# speedup vs baseline: 5.6648x; 5.6648x over previous
import functools

import jax
import jax.numpy as jnp
from jax import lax
from jax.experimental import pallas as pl
from jax.experimental.pallas import tpu as pltpu
from jax.experimental.pallas import tpu_sc as plsc

NC = 2
NS = 16
NW = NC * NS
LANES = 16
ECHUNK = 64

RATE_SCALE = 4.0
RATE_LO = 1.0


@functools.lru_cache(maxsize=None)
def _build(B, D, V):
    b_per_w = B // NW
    n_chunks = b_per_w // ECHUNK
    rows_per_view = D * V // 128
    sub_per_row = 128 // D

    mesh = plsc.VectorSubcoreMesh(core_axis_name="c", subcore_axis_name="s")

    @functools.partial(
        pl.kernel,
        mesh=mesh,
        out_type=jax.ShapeDtypeStruct((B,), jnp.float32),
        compiler_params=pltpu.CompilerParams(needs_layout_passes=False),
        scratch_types=[
            pltpu.VMEM((b_per_w,), jnp.int32),
            pltpu.VMEM((b_per_w,), jnp.int32),
            pltpu.VMEM((b_per_w,), jnp.int32),
            pltpu.VMEM((b_per_w,), jnp.int32),
            pltpu.VMEM((b_per_w,), jnp.int32),
            pltpu.VMEM((b_per_w,), jnp.int32),
            pltpu.VMEM((2, ECHUNK, 128), jnp.float32),
            pltpu.VMEM((2, ECHUNK, 128), jnp.float32),
            pltpu.VMEM((2, ECHUNK, 128), jnp.float32),
            pltpu.VMEM((2, ECHUNK, 128), jnp.float32),
            pltpu.VMEM((b_per_w,), jnp.float32),
            pltpu.SemaphoreType.DMA,
        ],
    )
    def mf_kernel(uidx_hbm, iidx_hbm, uview_hbm, iview_hbm, ub_hbm, ib_hbm,
                  out_hbm, uidx_v, iidx_v, urow_v, irow_v, ubrow_v, ibrow_v,
                  u_dst, v_dst, ub_dst, ib_dst, out_v, sem):
        wid = lax.axis_index("c") * NS + lax.axis_index("s")
        base_e = wid * b_per_w

        pltpu.sync_copy(uidx_hbm.at[pl.ds(base_e, b_per_w)], uidx_v)
        pltpu.sync_copy(iidx_hbm.at[pl.ds(base_e, b_per_w)], iidx_v)

        for k in range(b_per_w // LANES):
            s = pl.ds(k * LANES, LANES)
            ru = uidx_v[s]
            ri = iidx_v[s]
            urow_v[s] = ru >> 2
            irow_v[s] = ri >> 2
            ubrow_v[s] = ru >> 7
            ibrow_v[s] = ri >> 7

        def fire(k):
            p = k % 2
            s = pl.ds(k * ECHUNK, ECHUNK)
            pltpu.async_copy(uview_hbm.at[urow_v.at[s]], u_dst.at[p], sem)
            pltpu.async_copy(iview_hbm.at[irow_v.at[s]], v_dst.at[p], sem)
            pltpu.async_copy(ub_hbm.at[ubrow_v.at[s]], ub_dst.at[p], sem)
            pltpu.async_copy(ib_hbm.at[ibrow_v.at[s]], ib_dst.at[p], sem)

        def drain(k):
            p = k % 2
            pltpu.make_async_copy(
                uview_hbm.at[pl.ds(0, ECHUNK)], u_dst.at[p], sem).wait()
            pltpu.make_async_copy(
                uview_hbm.at[pl.ds(0, ECHUNK)], v_dst.at[p], sem).wait()
            pltpu.make_async_copy(
                ub_hbm.at[pl.ds(0, ECHUNK)], ub_dst.at[p], sem).wait()
            pltpu.make_async_copy(
                ub_hbm.at[pl.ds(0, ECHUNK)], ib_dst.at[p], sem).wait()

        def extract(k):
            p = k % 2
            pv = jnp.full((LANES,), p, jnp.int32)
            eidx = lax.iota(jnp.int32, LANES)
            for g in range(ECHUNK // LANES):
                s = pl.ds(k * ECHUNK + g * LANES, LANES)
                ru = uidx_v[s]
                ri = iidx_v[s]
                cu = (ru & (sub_per_row - 1)) * D
                ci = (ri & (sub_per_row - 1)) * D
                el = eidx + g * LANES
                acc = (plsc.load_gather(ub_dst, [pv, el, ru & 127])
                       + plsc.load_gather(ib_dst, [pv, el, ri & 127]))
                for c in range(D):
                    uc = plsc.load_gather(u_dst, [pv, el, cu + c])
                    vc = plsc.load_gather(v_dst, [pv, el, ci + c])
                    acc = acc + uc * vc
                out_v[s] = RATE_SCALE / (1.0 + jnp.exp(-acc)) + RATE_LO

        fire(0)
        for k in range(n_chunks):
            if k + 1 < n_chunks:
                fire(k + 1)
            drain(k)
            extract(k)

        pltpu.sync_copy(out_v, out_hbm.at[pl.ds(base_e, b_per_w)])

    return mf_kernel


def kernel(X, user_embeddings, user_bias, item_embeddings, item_bias):
    B = X.shape[0]
    V, D = user_embeddings.shape
    vpad = (-V) % 128
    uidx = X[:, 0].astype(jnp.int32)
    iidx = X[:, 1].astype(jnp.int32)
    uview = user_embeddings.reshape(V * D // 128, 128)
    iview = item_embeddings.reshape(V * D // 128, 128)
    ub = jnp.pad(user_bias.reshape(-1), (0, vpad)).reshape(-1, 128)
    ib = jnp.pad(item_bias.reshape(-1), (0, vpad)).reshape(-1, 128)
    out = _build(B, D, V)(uidx, iidx, uview, iview, ub, ib)
    return out.reshape(B, 1)


# speedup vs baseline: 15.1986x; 2.6830x over previous
import functools

import jax
import jax.numpy as jnp
from jax import lax
from jax.experimental import pallas as pl
from jax.experimental.pallas import tpu as pltpu
from jax.experimental.pallas import tpu_sc as plsc

NC = 2
NS = 16
NW = NC * NS
LANES = 16
EC = 4

RATE_SCALE = 4.0
RATE_LO = 1.0


@functools.lru_cache(maxsize=None)
def _build(B, D, V):
    b_per_w = B // NW
    n_chunks = b_per_w // EC
    idx_alloc = b_per_w + LANES
    max_tile = (V - 1) // 128

    mesh = plsc.VectorSubcoreMesh(core_axis_name="c", subcore_axis_name="s")

    @functools.partial(
        pl.kernel,
        mesh=mesh,
        out_type=jax.ShapeDtypeStruct((B,), jnp.float32),
        compiler_params=pltpu.CompilerParams(needs_layout_passes=False),
        scratch_types=[
            pltpu.VMEM((idx_alloc,), jnp.int32),
            pltpu.VMEM((idx_alloc,), jnp.int32),
            pltpu.VMEM((b_per_w,), jnp.int32),
            pltpu.VMEM((b_per_w,), jnp.int32),
            pltpu.VMEM((2, EC, D, 128), jnp.float32),
            pltpu.VMEM((2, EC, D, 128), jnp.float32),
            pltpu.VMEM((2, 64, 128), jnp.float32),
            pltpu.VMEM((2, 64, 128), jnp.float32),
            pltpu.VMEM((b_per_w,), jnp.float32),
            pltpu.VMEM((b_per_w,), jnp.float32),
            pltpu.VMEM((b_per_w,), jnp.float32),
            pltpu.SemaphoreType.DMA((2,)),
            pltpu.SemaphoreType.DMA((2,)),
        ],
    )
    def mf_kernel(uidx_hbm, iidx_hbm, uT_hbm, iT_hbm, ub_hbm, ib_hbm,
                  out_hbm, uidx_v, iidx_v, ubrow_v, ibrow_v, u_slabs,
                  v_slabs, ubb, ibb, ub_v, ib_v, out_v, sem, semb):
        wid = lax.axis_index("c") * NS + lax.axis_index("s")
        base_e = wid * b_per_w

        pltpu.sync_copy(uidx_hbm.at[pl.ds(base_e, b_per_w)],
                        uidx_v.at[pl.ds(0, b_per_w)])
        pltpu.sync_copy(iidx_hbm.at[pl.ds(base_e, b_per_w)],
                        iidx_v.at[pl.ds(0, b_per_w)])
        zeros = jnp.zeros((LANES,), jnp.int32)
        uidx_v[pl.ds(b_per_w, LANES)] = zeros
        iidx_v[pl.ds(b_per_w, LANES)] = zeros

        for kk in range(b_per_w // LANES):
            s = pl.ds(kk * LANES, LANES)
            ubrow_v[s] = uidx_v[s] >> 7
            ibrow_v[s] = iidx_v[s] >> 7

        eiota = lax.iota(jnp.int32, LANES)

        def fire_bias(j):
            p = j % 2
            s = pl.ds(j * 64, 64)
            pltpu.async_copy(ub_hbm.at[ubrow_v.at[s]], ubb.at[p], semb.at[p])
            pltpu.async_copy(ib_hbm.at[ibrow_v.at[s]], ibb.at[p], semb.at[p])

        def take_bias(j):
            p = j % 2
            pltpu.make_async_copy(ub_hbm.at[pl.ds(0, 64)], ubb.at[p],
                                  semb.at[p]).wait()
            pltpu.make_async_copy(ub_hbm.at[pl.ds(0, 64)], ibb.at[p],
                                  semb.at[p]).wait()
            pvb = jnp.full((LANES,), p, jnp.int32)
            for g in range(4):
                s = pl.ds(j * 64 + g * 16, LANES)
                elb = g * 16 + eiota
                mu = uidx_v[s] & 127
                mi = iidx_v[s] & 127
                ub_v[s] = plsc.load_gather(ubb, [pvb, elb, mu])
                ib_v[s] = plsc.load_gather(ibb, [pvb, elb, mi])

        fire_bias(0)
        for j in range(1, 9):
            if j < 8:
                fire_bias(j)
            take_bias(j - 1)

        def scalar_tile(vec, lane):
            t = jnp.sum(jnp.where(eiota == lane, vec >> 7, 0))
            return jnp.clip(t, 0, max_tile)

        def fire(k):
            p = k % 2
            tu = uidx_v[pl.ds(k * EC, LANES)]
            ti = iidx_v[pl.ds(k * EC, LANES)]
            for e in range(EC):
                off_u = pl.multiple_of(scalar_tile(tu, e) * 128, 128)
                off_v = pl.multiple_of(scalar_tile(ti, e) * 128, 128)
                pltpu.async_copy(uT_hbm.at[:, pl.ds(off_u, 128)],
                                 u_slabs.at[p, e], sem.at[p])
                pltpu.async_copy(iT_hbm.at[:, pl.ds(off_v, 128)],
                                 v_slabs.at[p, e], sem.at[p])

        def drain(k):
            p = k % 2
            for e in range(EC):
                pltpu.make_async_copy(uT_hbm.at[:, pl.ds(0, 128)],
                                      u_slabs.at[p, e], sem.at[p]).wait()
                pltpu.make_async_copy(uT_hbm.at[:, pl.ds(0, 128)],
                                      v_slabs.at[p, e], sem.at[p]).wait()

        def extract(k):
            p = k % 2
            pv = jnp.full((LANES,), 0, jnp.int32) + p
            el = eiota >> 2
            cl = eiota & 3
            ru = plsc.load_gather(uidx_v, [k * EC + el])
            ri = plsc.load_gather(iidx_v, [k * EC + el])
            mu = ru & 127
            mi = ri & 127
            acc = jnp.zeros((LANES,), jnp.float32)
            for cg in range(D // 4):
                c = cg * 4 + cl
                uc = plsc.load_gather(u_slabs, [pv, el, c, mu])
                vc = plsc.load_gather(v_slabs, [pv, el, c, mi])
                acc = acc + uc * vc
            acc = acc + jnp.take(acc, eiota ^ 1)
            acc = acc + jnp.take(acc, eiota ^ 2)
            bias = (plsc.load_gather(ub_v, [k * EC + el])
                    + plsc.load_gather(ib_v, [k * EC + el]))
            pred = RATE_SCALE / (1.0 + jnp.exp(-(acc + bias))) + RATE_LO
            plsc.store_scatter(out_v, [k * EC + el], pred,
                               mask=(eiota & 3) == 0)

        def body(k, carry):
            pl.when(k < n_chunks)(lambda: fire(k))

            def back():
                drain(k - 1)
                extract(k - 1)

            pl.when(k > 0)(back)
            return carry

        lax.fori_loop(0, n_chunks + 1, body, 0)

        pltpu.sync_copy(out_v, out_hbm.at[pl.ds(base_e, b_per_w)])

    return mf_kernel


def kernel(X, user_embeddings, user_bias, item_embeddings, item_bias):
    B = X.shape[0]
    V, D = user_embeddings.shape
    vpad = (-V) % 128
    uidx = X[:, 0].astype(jnp.int32)
    iidx = X[:, 1].astype(jnp.int32)
    ub = jnp.pad(user_bias.reshape(-1), (0, vpad)).reshape(-1, 128)
    ib = jnp.pad(item_bias.reshape(-1), (0, vpad)).reshape(-1, 128)
    out = _build(B, D, V)(uidx, iidx, user_embeddings.T, item_embeddings.T,
                          ub, ib)
    return out.reshape(B, 1)


# speedup vs baseline: 15.9482x; 1.0493x over previous
import functools

import jax
import jax.numpy as jnp
from jax import lax
from jax.experimental import pallas as pl
from jax.experimental.pallas import tpu as pltpu
from jax.experimental.pallas import tpu_sc as plsc

NC = 2
NS = 16
NW = NC * NS
LANES = 16
EC = 4
NBUF = 3

RATE_SCALE = 4.0
RATE_LO = 1.0


@functools.lru_cache(maxsize=None)
def _build(B, D, V):
    b_per_w = B // NW
    n_chunks = b_per_w // EC
    idx_alloc = b_per_w + LANES
    max_tile = (V - 1) // 128

    mesh = plsc.VectorSubcoreMesh(core_axis_name="c", subcore_axis_name="s")

    @functools.partial(
        pl.kernel,
        mesh=mesh,
        out_type=jax.ShapeDtypeStruct((B,), jnp.float32),
        compiler_params=pltpu.CompilerParams(needs_layout_passes=False),
        scratch_types=[
            pltpu.VMEM((idx_alloc,), jnp.int32),
            pltpu.VMEM((idx_alloc,), jnp.int32),
            pltpu.VMEM((b_per_w,), jnp.int32),
            pltpu.VMEM((b_per_w,), jnp.int32),
            pltpu.VMEM((NBUF, EC, D, 128), jnp.float32),
            pltpu.VMEM((NBUF, EC, D, 128), jnp.float32),
            pltpu.VMEM((2, 32, 128), jnp.float32),
            pltpu.VMEM((2, 32, 128), jnp.float32),
            pltpu.VMEM((b_per_w,), jnp.float32),
            pltpu.VMEM((b_per_w,), jnp.float32),
            pltpu.VMEM((b_per_w,), jnp.float32),
            pltpu.SemaphoreType.DMA((NBUF,)),
            pltpu.SemaphoreType.DMA((2,)),
        ],
    )
    def mf_kernel(uidx_hbm, iidx_hbm, uT_hbm, iT_hbm, ub_hbm, ib_hbm,
                  out_hbm, uidx_v, iidx_v, ubrow_v, ibrow_v, u_slabs,
                  v_slabs, ubb, ibb, ub_v, ib_v, out_v, sem, semb):
        wid = lax.axis_index("c") * NS + lax.axis_index("s")
        base_e = wid * b_per_w

        pltpu.sync_copy(uidx_hbm.at[pl.ds(base_e, b_per_w)],
                        uidx_v.at[pl.ds(0, b_per_w)])
        pltpu.sync_copy(iidx_hbm.at[pl.ds(base_e, b_per_w)],
                        iidx_v.at[pl.ds(0, b_per_w)])
        zeros = jnp.zeros((LANES,), jnp.int32)
        uidx_v[pl.ds(b_per_w, LANES)] = zeros
        iidx_v[pl.ds(b_per_w, LANES)] = zeros

        for kk in range(b_per_w // LANES):
            s = pl.ds(kk * LANES, LANES)
            ubrow_v[s] = uidx_v[s] >> 7
            ibrow_v[s] = iidx_v[s] >> 7

        eiota = lax.iota(jnp.int32, LANES)

        def fire_bias(j):
            p = j % 2
            s = pl.ds(j * 32, 32)
            pltpu.async_copy(ub_hbm.at[ubrow_v.at[s]], ubb.at[p], semb.at[p])
            pltpu.async_copy(ib_hbm.at[ibrow_v.at[s]], ibb.at[p], semb.at[p])

        def take_bias(j):
            p = j % 2
            pltpu.make_async_copy(ub_hbm.at[pl.ds(0, 32)], ubb.at[p],
                                  semb.at[p]).wait()
            pltpu.make_async_copy(ub_hbm.at[pl.ds(0, 32)], ibb.at[p],
                                  semb.at[p]).wait()
            pvb = jnp.full((LANES,), p, jnp.int32)
            for g in range(2):
                s = pl.ds(j * 32 + g * 16, LANES)
                elb = g * 16 + eiota
                mu = uidx_v[s] & 127
                mi = iidx_v[s] & 127
                ub_v[s] = plsc.load_gather(ubb, [pvb, elb, mu])
                ib_v[s] = plsc.load_gather(ibb, [pvb, elb, mi])

        fire_bias(0)
        for j in range(1, 17):
            if j < 16:
                fire_bias(j)
            take_bias(j - 1)

        def scalar_tile(vec, lane):
            t = jnp.sum(jnp.where(eiota == lane, vec >> 7, 0))
            return jnp.clip(t, 0, max_tile)

        def fire(k):
            p = k % NBUF
            tu = uidx_v[pl.ds(k * EC, LANES)]
            ti = iidx_v[pl.ds(k * EC, LANES)]
            for e in range(EC):
                off_u = pl.multiple_of(scalar_tile(tu, e) * 128, 128)
                off_v = pl.multiple_of(scalar_tile(ti, e) * 128, 128)
                pltpu.async_copy(uT_hbm.at[:, pl.ds(off_u, 128)],
                                 u_slabs.at[p, e], sem.at[p])
                pltpu.async_copy(iT_hbm.at[:, pl.ds(off_v, 128)],
                                 v_slabs.at[p, e], sem.at[p])

        def drain(k):
            p = k % NBUF
            for e in range(EC):
                pltpu.make_async_copy(uT_hbm.at[:, pl.ds(0, 128)],
                                      u_slabs.at[p, e], sem.at[p]).wait()
                pltpu.make_async_copy(uT_hbm.at[:, pl.ds(0, 128)],
                                      v_slabs.at[p, e], sem.at[p]).wait()

        def extract(k):
            p = k % NBUF
            pv = jnp.full((LANES,), 0, jnp.int32) + p
            el = eiota >> 2
            cl = eiota & 3
            ru = plsc.load_gather(uidx_v, [k * EC + el])
            ri = plsc.load_gather(iidx_v, [k * EC + el])
            mu = ru & 127
            mi = ri & 127
            acc = jnp.zeros((LANES,), jnp.float32)
            for cg in range(D // 4):
                c = cg * 4 + cl
                uc = plsc.load_gather(u_slabs, [pv, el, c, mu])
                vc = plsc.load_gather(v_slabs, [pv, el, c, mi])
                acc = acc + uc * vc
            acc = acc + jnp.take(acc, eiota ^ 1)
            acc = acc + jnp.take(acc, eiota ^ 2)
            bias = (plsc.load_gather(ub_v, [k * EC + el])
                    + plsc.load_gather(ib_v, [k * EC + el]))
            pred = RATE_SCALE / (1.0 + jnp.exp(-(acc + bias))) + RATE_LO
            plsc.store_scatter(out_v, [k * EC + el], pred,
                               mask=(eiota & 3) == 0)

        lag = NBUF - 1

        def body(k, carry):
            pl.when(k < n_chunks)(lambda: fire(k))

            def back():
                drain(k - lag)
                extract(k - lag)

            pl.when(k >= lag)(back)
            return carry

        lax.fori_loop(0, n_chunks + lag, body, 0)

        pltpu.sync_copy(out_v, out_hbm.at[pl.ds(base_e, b_per_w)])

    return mf_kernel


def kernel(X, user_embeddings, user_bias, item_embeddings, item_bias):
    B = X.shape[0]
    V, D = user_embeddings.shape
    vpad = (-V) % 128
    uidx = X[:, 0].astype(jnp.int32)
    iidx = X[:, 1].astype(jnp.int32)
    ub = jnp.pad(user_bias.reshape(-1), (0, vpad)).reshape(-1, 128)
    ib = jnp.pad(item_bias.reshape(-1), (0, vpad)).reshape(-1, 128)
    out = _build(B, D, V)(uidx, iidx, user_embeddings.T, item_embeddings.T,
                          ub, ib)
    return out.reshape(B, 1)


# speedup vs baseline: 16.0752x; 1.0080x over previous
import functools

import jax
import jax.numpy as jnp
from jax import lax
from jax.experimental import pallas as pl
from jax.experimental.pallas import tpu as pltpu
from jax.experimental.pallas import tpu_sc as plsc

NC = 2
NS = 16
NW = NC * NS
LANES = 16
EC = 2
NBUF = 4

RATE_SCALE = 4.0
RATE_LO = 1.0


@functools.lru_cache(maxsize=None)
def _build(B, D, V):
    b_per_w = B // NW
    n_chunks = b_per_w // EC
    idx_alloc = b_per_w + LANES
    max_tile = (V - 1) // 128

    mesh = plsc.VectorSubcoreMesh(core_axis_name="c", subcore_axis_name="s")

    @functools.partial(
        pl.kernel,
        mesh=mesh,
        out_type=jax.ShapeDtypeStruct((B,), jnp.float32),
        compiler_params=pltpu.CompilerParams(needs_layout_passes=False),
        scratch_types=[
            pltpu.VMEM((idx_alloc,), jnp.int32),
            pltpu.VMEM((idx_alloc,), jnp.int32),
            pltpu.VMEM((b_per_w,), jnp.int32),
            pltpu.VMEM((b_per_w,), jnp.int32),
            pltpu.VMEM((NBUF, EC, D, 128), jnp.float32),
            pltpu.VMEM((NBUF, EC, D, 128), jnp.float32),
            pltpu.VMEM((2, 32, 128), jnp.float32),
            pltpu.VMEM((2, 32, 128), jnp.float32),
            pltpu.VMEM((b_per_w,), jnp.float32),
            pltpu.VMEM((b_per_w,), jnp.float32),
            pltpu.VMEM((b_per_w,), jnp.float32),
            pltpu.SemaphoreType.DMA((NBUF,)),
            pltpu.SemaphoreType.DMA((2,)),
        ],
    )
    def mf_kernel(uidx_hbm, iidx_hbm, uT_hbm, iT_hbm, ub_hbm, ib_hbm,
                  out_hbm, uidx_v, iidx_v, ubrow_v, ibrow_v, u_slabs,
                  v_slabs, ubb, ibb, ub_v, ib_v, out_v, sem, semb):
        wid = lax.axis_index("c") * NS + lax.axis_index("s")
        base_e = wid * b_per_w

        pltpu.sync_copy(uidx_hbm.at[pl.ds(base_e, b_per_w)],
                        uidx_v.at[pl.ds(0, b_per_w)])
        pltpu.sync_copy(iidx_hbm.at[pl.ds(base_e, b_per_w)],
                        iidx_v.at[pl.ds(0, b_per_w)])
        zeros = jnp.zeros((LANES,), jnp.int32)
        uidx_v[pl.ds(b_per_w, LANES)] = zeros
        iidx_v[pl.ds(b_per_w, LANES)] = zeros

        for kk in range(b_per_w // LANES):
            s = pl.ds(kk * LANES, LANES)
            ubrow_v[s] = uidx_v[s] >> 7
            ibrow_v[s] = iidx_v[s] >> 7

        eiota = lax.iota(jnp.int32, LANES)

        def fire_bias(j):
            p = j % 2
            s = pl.ds(j * 32, 32)
            pltpu.async_copy(ub_hbm.at[ubrow_v.at[s]], ubb.at[p], semb.at[p])
            pltpu.async_copy(ib_hbm.at[ibrow_v.at[s]], ibb.at[p], semb.at[p])

        def take_bias(j):
            p = j % 2
            pltpu.make_async_copy(ub_hbm.at[pl.ds(0, 32)], ubb.at[p],
                                  semb.at[p]).wait()
            pltpu.make_async_copy(ub_hbm.at[pl.ds(0, 32)], ibb.at[p],
                                  semb.at[p]).wait()
            pvb = jnp.full((LANES,), p, jnp.int32)
            for g in range(2):
                s = pl.ds(j * 32 + g * 16, LANES)
                elb = g * 16 + eiota
                mu = uidx_v[s] & 127
                mi = iidx_v[s] & 127
                ub_v[s] = plsc.load_gather(ubb, [pvb, elb, mu])
                ib_v[s] = plsc.load_gather(ibb, [pvb, elb, mi])

        fire_bias(0)
        for j in range(1, 17):
            if j < 16:
                fire_bias(j)
            take_bias(j - 1)

        def scalar_tile(vec, lane):
            t = jnp.sum(jnp.where(eiota == lane, vec >> 7, 0))
            return jnp.clip(t, 0, max_tile)

        def fire(k):
            p = k % NBUF
            kbase = (k * EC // LANES) * LANES
            tu = uidx_v[pl.ds(kbase, LANES)]
            ti = iidx_v[pl.ds(kbase, LANES)]
            for e in range(EC):
                lane = k * EC - kbase + e
                off_u = pl.multiple_of(scalar_tile(tu, lane) * 128, 128)
                off_v = pl.multiple_of(scalar_tile(ti, lane) * 128, 128)
                pltpu.async_copy(uT_hbm.at[:, pl.ds(off_u, 128)],
                                 u_slabs.at[p, e], sem.at[p])
                pltpu.async_copy(iT_hbm.at[:, pl.ds(off_v, 128)],
                                 v_slabs.at[p, e], sem.at[p])

        def drain(k):
            p = k % NBUF
            for e in range(EC):
                pltpu.make_async_copy(uT_hbm.at[:, pl.ds(0, 128)],
                                      u_slabs.at[p, e], sem.at[p]).wait()
                pltpu.make_async_copy(uT_hbm.at[:, pl.ds(0, 128)],
                                      v_slabs.at[p, e], sem.at[p]).wait()

        def extract(k):
            p = k % NBUF
            pv = jnp.full((LANES,), 0, jnp.int32) + p
            lpe = LANES // EC
            el = eiota // lpe
            cl = eiota % lpe
            ru = plsc.load_gather(uidx_v, [k * EC + el])
            ri = plsc.load_gather(iidx_v, [k * EC + el])
            mu = ru & 127
            mi = ri & 127
            acc = jnp.zeros((LANES,), jnp.float32)
            for cg in range(D // lpe):
                c = cg * lpe + cl
                uc = plsc.load_gather(u_slabs, [pv, el, c, mu])
                vc = plsc.load_gather(v_slabs, [pv, el, c, mi])
                acc = acc + uc * vc
            step = 1
            while step < lpe:
                acc = acc + jnp.take(acc, eiota ^ step)
                step *= 2
            bias = (plsc.load_gather(ub_v, [k * EC + el])
                    + plsc.load_gather(ib_v, [k * EC + el]))
            pred = RATE_SCALE / (1.0 + jnp.exp(-(acc + bias))) + RATE_LO
            plsc.store_scatter(out_v, [k * EC + el], pred,
                               mask=cl == 0)

        lag = NBUF - 1

        def body(k, carry):
            pl.when(k < n_chunks)(lambda: fire(k))

            def back():
                drain(k - lag)
                extract(k - lag)

            pl.when(k >= lag)(back)
            return carry

        lax.fori_loop(0, n_chunks + lag, body, 0)

        pltpu.sync_copy(out_v, out_hbm.at[pl.ds(base_e, b_per_w)])

    return mf_kernel


def kernel(X, user_embeddings, user_bias, item_embeddings, item_bias):
    B = X.shape[0]
    V, D = user_embeddings.shape
    vpad = (-V) % 128
    uidx = X[:, 0].astype(jnp.int32)
    iidx = X[:, 1].astype(jnp.int32)
    ub = jnp.pad(user_bias.reshape(-1), (0, vpad)).reshape(-1, 128)
    ib = jnp.pad(item_bias.reshape(-1), (0, vpad)).reshape(-1, 128)
    out = _build(B, D, V)(uidx, iidx, user_embeddings.T, item_embeddings.T,
                          ub, ib)
    return out.reshape(B, 1)


# speedup vs baseline: 17.0482x; 1.0605x over previous
import functools

import jax
import jax.numpy as jnp
from jax import lax
from jax.experimental import pallas as pl
from jax.experimental.pallas import tpu as pltpu
from jax.experimental.pallas import tpu_sc as plsc

NC = 2
NS = 16
NW = NC * NS
LANES = 16
EC = 2
NBUF = 6

RATE_SCALE = 4.0
RATE_LO = 1.0


@functools.lru_cache(maxsize=None)
def _build(B, D, V):
    b_per_w = B // NW
    n_chunks = b_per_w // EC
    idx_alloc = b_per_w + LANES
    max_tile = (V - 1) // 128

    mesh = plsc.VectorSubcoreMesh(core_axis_name="c", subcore_axis_name="s")

    @functools.partial(
        pl.kernel,
        mesh=mesh,
        out_type=jax.ShapeDtypeStruct((B,), jnp.float32),
        compiler_params=pltpu.CompilerParams(needs_layout_passes=False),
        scratch_types=[
            pltpu.VMEM((idx_alloc,), jnp.int32),
            pltpu.VMEM((idx_alloc,), jnp.int32),
            pltpu.VMEM((b_per_w,), jnp.int32),
            pltpu.VMEM((b_per_w,), jnp.int32),
            pltpu.VMEM((NBUF, EC, D, 128), jnp.float32),
            pltpu.VMEM((NBUF, EC, D, 128), jnp.float32),
            pltpu.VMEM((2, 32, 128), jnp.float32),
            pltpu.VMEM((2, 32, 128), jnp.float32),
            pltpu.VMEM((b_per_w,), jnp.float32),
            pltpu.VMEM((b_per_w,), jnp.float32),
            pltpu.VMEM((b_per_w,), jnp.float32),
            pltpu.SemaphoreType.DMA((NBUF,)),
            pltpu.SemaphoreType.DMA((2,)),
        ],
    )
    def mf_kernel(uidx_hbm, iidx_hbm, uT_hbm, iT_hbm, ub_hbm, ib_hbm,
                  out_hbm, uidx_v, iidx_v, ubrow_v, ibrow_v, u_slabs,
                  v_slabs, ubb, ibb, ub_v, ib_v, out_v, sem, semb):
        wid = lax.axis_index("c") * NS + lax.axis_index("s")
        base_e = wid * b_per_w

        pltpu.sync_copy(uidx_hbm.at[pl.ds(base_e, b_per_w)],
                        uidx_v.at[pl.ds(0, b_per_w)])
        pltpu.sync_copy(iidx_hbm.at[pl.ds(base_e, b_per_w)],
                        iidx_v.at[pl.ds(0, b_per_w)])
        zeros = jnp.zeros((LANES,), jnp.int32)
        uidx_v[pl.ds(b_per_w, LANES)] = zeros
        iidx_v[pl.ds(b_per_w, LANES)] = zeros

        for kk in range(b_per_w // LANES):
            s = pl.ds(kk * LANES, LANES)
            ubrow_v[s] = uidx_v[s] >> 7
            ibrow_v[s] = iidx_v[s] >> 7

        eiota = lax.iota(jnp.int32, LANES)

        def fire_bias(j):
            p = j % 2
            s = pl.ds(j * 32, 32)
            pltpu.async_copy(ub_hbm.at[ubrow_v.at[s]], ubb.at[p], semb.at[p])
            pltpu.async_copy(ib_hbm.at[ibrow_v.at[s]], ibb.at[p], semb.at[p])

        def take_bias(j):
            p = j % 2
            pltpu.make_async_copy(ub_hbm.at[pl.ds(0, 32)], ubb.at[p],
                                  semb.at[p]).wait()
            pltpu.make_async_copy(ub_hbm.at[pl.ds(0, 32)], ibb.at[p],
                                  semb.at[p]).wait()
            pvb = jnp.full((LANES,), p, jnp.int32)
            for g in range(2):
                s = pl.ds(j * 32 + g * 16, LANES)
                elb = g * 16 + eiota
                mu = uidx_v[s] & 127
                mi = iidx_v[s] & 127
                ub_v[s] = plsc.load_gather(ubb, [pvb, elb, mu])
                ib_v[s] = plsc.load_gather(ibb, [pvb, elb, mi])

        fire_bias(0)
        for j in range(1, 17):
            if j < 16:
                fire_bias(j)
            take_bias(j - 1)

        def scalar_tile(vec, lane):
            t = jnp.sum(jnp.where(eiota == lane, vec >> 7, 0))
            return jnp.clip(t, 0, max_tile)

        def fire(k):
            p = k % NBUF
            kbase = (k * EC // LANES) * LANES
            tu = uidx_v[pl.ds(kbase, LANES)]
            ti = iidx_v[pl.ds(kbase, LANES)]
            for e in range(EC):
                lane = k * EC - kbase + e
                off_u = pl.multiple_of(scalar_tile(tu, lane) * 128, 128)
                off_v = pl.multiple_of(scalar_tile(ti, lane) * 128, 128)
                pltpu.async_copy(uT_hbm.at[:, pl.ds(off_u, 128)],
                                 u_slabs.at[p, e], sem.at[p])
                pltpu.async_copy(iT_hbm.at[:, pl.ds(off_v, 128)],
                                 v_slabs.at[p, e], sem.at[p])

        def drain(k):
            p = k % NBUF
            for e in range(EC):
                pltpu.make_async_copy(uT_hbm.at[:, pl.ds(0, 128)],
                                      u_slabs.at[p, e], sem.at[p]).wait()
                pltpu.make_async_copy(uT_hbm.at[:, pl.ds(0, 128)],
                                      v_slabs.at[p, e], sem.at[p]).wait()

        def extract(k):
            p = k % NBUF
            pv = jnp.full((LANES,), 0, jnp.int32) + p
            lpe = LANES // EC
            el = eiota // lpe
            cl = eiota % lpe
            ru = plsc.load_gather(uidx_v, [k * EC + el])
            ri = plsc.load_gather(iidx_v, [k * EC + el])
            mu = ru & 127
            mi = ri & 127
            acc = jnp.zeros((LANES,), jnp.float32)
            for cg in range(D // lpe):
                c = cg * lpe + cl
                uc = plsc.load_gather(u_slabs, [pv, el, c, mu])
                vc = plsc.load_gather(v_slabs, [pv, el, c, mi])
                acc = acc + uc * vc
            step = 1
            while step < lpe:
                acc = acc + jnp.take(acc, eiota ^ step)
                step *= 2
            bias = (plsc.load_gather(ub_v, [k * EC + el])
                    + plsc.load_gather(ib_v, [k * EC + el]))
            pred = RATE_SCALE / (1.0 + jnp.exp(-(acc + bias))) + RATE_LO
            plsc.store_scatter(out_v, [k * EC + el], pred,
                               mask=cl == 0)

        lag = NBUF - 1

        def body(k, carry):
            pl.when(k < n_chunks)(lambda: fire(k))

            def back():
                drain(k - lag)
                extract(k - lag)

            pl.when(k >= lag)(back)
            return carry

        lax.fori_loop(0, n_chunks + lag, body, 0)

        pltpu.sync_copy(out_v, out_hbm.at[pl.ds(base_e, b_per_w)])

    return mf_kernel


def kernel(X, user_embeddings, user_bias, item_embeddings, item_bias):
    B = X.shape[0]
    V, D = user_embeddings.shape
    vpad = (-V) % 128
    uidx = X[:, 0].astype(jnp.int32)
    iidx = X[:, 1].astype(jnp.int32)
    ub = jnp.pad(user_bias.reshape(-1), (0, vpad)).reshape(-1, 128)
    ib = jnp.pad(item_bias.reshape(-1), (0, vpad)).reshape(-1, 128)
    out = _build(B, D, V)(uidx, iidx, user_embeddings.T, item_embeddings.T,
                          ub, ib)
    return out.reshape(B, 1)
